# Initial kernel scaffold; baseline (speedup 1.0000x reference)
import jax
import jax.numpy as jnp
from jax.experimental import pallas as pl


def kernel(pos, x, batch, edge_index, node_table, Wq, Wk, Wv, We, Wo, ln1_s, ln1_b, W1, b1, W2, b2, ln2_s, ln2_b, RW1, rb1, RW2, rb2):
    raise NotImplementedError("write your pallas kernel here")


# speedup vs baseline: 2.8670x; 2.8670x over previous
import functools

import jax
import jax.numpy as jnp
from jax import lax
from jax.experimental import pallas as pl
from jax.experimental.pallas import tpu as pltpu
from jax.experimental.pallas import tpu_sc as plsc

_N = 10000
_NP = 10112
_NBLK = _NP // 128
_E = 320000
_D = 128
_NW = 32
_EPW = _E // _NW
_GCH = 80
_GIT = _EPW // _GCH
_RPT = _NP // 16
_EB = 1280
_EG = _E // _EB
_CUTON = 3.0
_CUTOFF = 8.0
_WIDTH = (_CUTOFF - _CUTON) / (_D - 1)


def _mesh():
  return plsc.VectorSubcoreMesh(core_axis_name="c", subcore_axis_name="s")


def _sc_gather(tabs, idxs):
  n = len(tabs)
  width = tabs[0].shape[1]

  @functools.partial(
      pl.kernel,
      mesh=_mesh(),
      out_type=[jax.ShapeDtypeStruct((_E, width), jnp.float32)
                for _ in range(n)],
      scratch_types=[
          pltpu.VMEM((1, _GCH), jnp.int32),
          pltpu.VMEM((_GCH, width), jnp.float32),
          pltpu.SemaphoreType.DMA,
      ],
  )
  def gk(*refs):
    tab_r = refs[:n]
    idx_r = refs[n:2 * n]
    out_r = refs[2 * n:3 * n]
    ixv, buf, sem = refs[3 * n:]
    c = lax.axis_index("c")
    s = lax.axis_index("s")
    w = s * 2 + c
    base = w * _EPW

    def body(j, carry):
      off = base + j * _GCH
      for t in range(n):
        pltpu.sync_copy(idx_r[t].at[pl.ds(off, _GCH)], ixv.at[0])
        pltpu.async_copy(tab_r[t].at[ixv.at[0]], buf, sem).wait()
        pltpu.sync_copy(buf, out_r[t].at[pl.ds(off, _GCH)])
      return carry

    lax.fori_loop(0, _GIT, body, 0)

  return gk(*tabs, *idxs)


def _sc_scatter(msg, exd, row, znm):
  epw = _E // 16
  its = epw // _GCH

  @functools.partial(
      pl.kernel,
      mesh=_mesh(),
      out_type=[
          jax.ShapeDtypeStruct((_NP, 128), jnp.float32),
          jax.ShapeDtypeStruct((_NP, 128), jnp.float32),
      ],
      scratch_types=[
          pltpu.VMEM((1, _GCH), jnp.int32),
          pltpu.VMEM((_GCH, 128), jnp.float32),
          pltpu.VMEM_SHARED((_NP, 128), jnp.float32),
          pltpu.SemaphoreType.DMA,
      ],
  )
  def sk(msg_h, exd_h, row_h, znm_h, nm_o, dn_o, ixv, mb, acc, sem):
    c = lax.axis_index("c")
    s = lax.axis_index("s")
    r0 = s * _RPT
    pltpu.sync_copy(znm_h.at[pl.ds(r0, _RPT)], acc.at[pl.ds(r0, _RPT)])
    plsc.subcore_barrier()
    base = s * epw

    def body(src_h):
      def step(j, carry):
        off = base + j * _GCH
        pltpu.sync_copy(row_h.at[pl.ds(off, _GCH)], ixv.at[0])
        pltpu.sync_copy(src_h.at[pl.ds(off, _GCH)], mb)
        pltpu.sync_copy(mb, acc.at[ixv.at[0]], add=True)
        return carry
      lax.fori_loop(0, its, step, 0)

    @pl.when(c == 0)
    def _():
      body(msg_h)

    @pl.when(c == 1)
    def _():
      body(exd_h)

    plsc.subcore_barrier()

    @pl.when(c == 0)
    def _():
      pltpu.sync_copy(acc.at[pl.ds(r0, _RPT)], nm_o.at[pl.ds(r0, _RPT)])

    @pl.when(c == 1)
    def _():
      pltpu.sync_copy(acc.at[pl.ds(r0, _RPT)], dn_o.at[pl.ds(r0, _RPT)])

  return sk(msg, exd, row, znm)


def _tc_embed(x2, table_pad):
  def body(x_ref, t_ref, o_ref):
    xi = x_ref[...]
    oh = (xi == lax.broadcasted_iota(jnp.int32, (128, 128), 1)
          ).astype(jnp.float32)
    o_ref[...] = jnp.dot(oh, t_ref[...], preferred_element_type=jnp.float32, precision=lax.Precision.HIGHEST)

  return pl.pallas_call(
      body,
      grid=(_NBLK,),
      in_specs=[
          pl.BlockSpec((128, 1), lambda i: (i, 0)),
          pl.BlockSpec((128, 128), lambda i: (0, 0)),
      ],
      out_specs=pl.BlockSpec((128, 128), lambda i: (i, 0)),
      out_shape=jax.ShapeDtypeStruct((_NP, 128), jnp.float32),
  )(x2, table_pad)


def _tc_qkv(h, wq, wk, wv):
  def body(h_ref, q_w, k_w, v_w, q_o, k_o, v_o):
    hb = h_ref[...]
    q_o[...] = jnp.dot(hb, q_w[...], preferred_element_type=jnp.float32, precision=lax.Precision.HIGHEST)
    k_o[...] = jnp.dot(hb, k_w[...], preferred_element_type=jnp.float32, precision=lax.Precision.HIGHEST)
    v_o[...] = jnp.dot(hb, v_w[...], preferred_element_type=jnp.float32, precision=lax.Precision.HIGHEST)

  wspec = pl.BlockSpec((128, 128), lambda i: (0, 0))
  hspec = pl.BlockSpec((128, 128), lambda i: (i, 0))
  return pl.pallas_call(
      body,
      grid=(_NBLK,),
      in_specs=[hspec, wspec, wspec, wspec],
      out_specs=[hspec, hspec, hspec],
      out_shape=[jax.ShapeDtypeStruct((_NP, 128), jnp.float32)] * 3,
  )(h, wq, wk, wv)


def _tc_elen(posr, posc):
  def body(pr, pc, el_o):
    d = pr[...] - pc[...]
    d2 = jnp.sum(d * d, axis=1, keepdims=True)
    el_o[...] = jnp.sqrt(d2)

  espec = pl.BlockSpec((_EB, 128), lambda i: (i, 0))
  return pl.pallas_call(
      body,
      grid=(_EG,),
      in_specs=[espec, espec],
      out_specs=pl.BlockSpec((_EB, 1), lambda i: (i, 0)),
      out_shape=jax.ShapeDtypeStruct((_E, 1), jnp.float32),
  )(posr, posc)


def _tc_edge(elen, qr, kc, vc, we_l, offs):
  def body(el_r, q_r, k_r, v_r, w_r, of_r, msg_o, exd_o):
    el = el_r[...]
    a = (el - of_r[...]) * (1.0 / _WIDTH)
    attr = jnp.exp(-0.5 * a * a)
    e = jnp.dot(attr, w_r[...], preferred_element_type=jnp.float32, precision=lax.Precision.HIGHEST)
    prod = q_r[...] * k_r[...] * e
    r16 = lax.broadcasted_iota(jnp.int32, (128, 16), 0) // 16
    c16 = lax.broadcasted_iota(jnp.int32, (128, 16), 1)
    seg = (r16 == c16).astype(jnp.float32)
    s16 = jnp.dot(prod, seg, preferred_element_type=jnp.float32, precision=lax.Precision.HIGHEST) * 0.25
    ex = jnp.exp(s16)
    rt = lax.broadcasted_iota(jnp.int32, (16, 128), 0)
    ct = lax.broadcasted_iota(jnp.int32, (16, 128), 1) // 16
    segt = (rt == ct).astype(jnp.float32)
    exd = jnp.dot(ex, segt, preferred_element_type=jnp.float32, precision=lax.Precision.HIGHEST)
    exd_o[...] = exd
    msg_o[...] = exd * v_r[...]

  espec = pl.BlockSpec((_EB, 128), lambda i: (i, 0))
  return pl.pallas_call(
      body,
      grid=(_EG,),
      in_specs=[
          pl.BlockSpec((_EB, 1), lambda i: (i, 0)),
          espec, espec, espec,
          pl.BlockSpec((128, 128), lambda i: (0, 0)),
          pl.BlockSpec((1, 128), lambda i: (0, 0)),
      ],
      out_specs=[espec, espec],
      out_shape=[
          jax.ShapeDtypeStruct((_E, 128), jnp.float32),
          jax.ShapeDtypeStruct((_E, 128), jnp.float32),
      ],
  )(elen, qr, kc, vc, we_l, offs)


def _layernorm(t, s_row, b_row):
  mu = jnp.mean(t, axis=-1, keepdims=True)
  var = jnp.mean((t - mu) * (t - mu), axis=-1, keepdims=True)
  return (t - mu) / jnp.sqrt(var + 1e-5) * s_row + b_row


def _tc_update(h, nm, dnb, wo, l1s, l1b, w1, b1, w2, b2, l2s, l2b):
  def body(h_ref, n_r, d_r, wo_r, l1s_r, l1b_r, w1_r, b1_r,
           w2_r, b2_r, l2s_r, l2b_r, o_ref):
    agg = n_r[...] / (d_r[...] + 1e-9)
    t1 = _layernorm(
        h_ref[...] + jnp.dot(agg, wo_r[...],
                             preferred_element_type=jnp.float32, precision=lax.Precision.HIGHEST),
        l1s_r[...], l1b_r[...])
    ff = jnp.dot(
        jnp.maximum(
            jnp.dot(t1, w1_r[...], preferred_element_type=jnp.float32, precision=lax.Precision.HIGHEST)
            + b1_r[...], 0.0),
        w2_r[...], preferred_element_type=jnp.float32, precision=lax.Precision.HIGHEST) + b2_r[...]
    o_ref[...] = _layernorm(t1 + ff, l2s_r[...], l2b_r[...])

  hspec = pl.BlockSpec((128, 128), lambda i: (i, 0))
  return pl.pallas_call(
      body,
      grid=(_NBLK,),
      in_specs=[
          hspec,
          hspec,
          hspec,
          pl.BlockSpec((128, 128), lambda i: (0, 0)),
          pl.BlockSpec((1, 128), lambda i: (0, 0)),
          pl.BlockSpec((1, 128), lambda i: (0, 0)),
          pl.BlockSpec((128, 256), lambda i: (0, 0)),
          pl.BlockSpec((1, 256), lambda i: (0, 0)),
          pl.BlockSpec((256, 128), lambda i: (0, 0)),
          pl.BlockSpec((1, 128), lambda i: (0, 0)),
          pl.BlockSpec((1, 128), lambda i: (0, 0)),
          pl.BlockSpec((1, 128), lambda i: (0, 0)),
      ],
      out_specs=hspec,
      out_shape=jax.ShapeDtypeStruct((_NP, 128), jnp.float32),
  )(h, nm, dnb, wo, l1s, l1b, w1, b1, w2, b2, l2s, l2b)


def _tc_pool(batch3, h, rw1, rb1r, rw2p, rb2p):
  def body(b_ref, h_ref, w1_r, b1_r, w2_r, b2_r, o_ref, acc):
    i = pl.program_id(0)

    @pl.when(i == 0)
    def _():
      acc[...] = jnp.zeros_like(acc)

    bv = b_ref[0]
    oh = (jnp.broadcast_to(bv, (16, 128))
          == lax.broadcasted_iota(jnp.int32, (16, 128), 0)
          ).astype(jnp.float32)
    acc[...] += jnp.dot(oh, h_ref[...], preferred_element_type=jnp.float32, precision=lax.Precision.HIGHEST)

    @pl.when(i == _NBLK - 1)
    def _():
      z = jnp.maximum(
          jnp.dot(acc[...], w1_r[...], preferred_element_type=jnp.float32, precision=lax.Precision.HIGHEST)
          + b1_r[...], 0.0)
      o_ref[...] = jnp.dot(
          z, w2_r[...], preferred_element_type=jnp.float32, precision=lax.Precision.HIGHEST) + b2_r[...]

  return pl.pallas_call(
      body,
      grid=(_NBLK,),
      in_specs=[
          pl.BlockSpec((1, 1, 128), lambda i: (i, 0, 0)),
          pl.BlockSpec((128, 128), lambda i: (i, 0)),
          pl.BlockSpec((128, 128), lambda i: (0, 0)),
          pl.BlockSpec((1, 128), lambda i: (0, 0)),
          pl.BlockSpec((128, 128), lambda i: (0, 0)),
          pl.BlockSpec((1, 128), lambda i: (0, 0)),
      ],
      out_specs=pl.BlockSpec((16, 128), lambda i: (0, 0)),
      out_shape=jax.ShapeDtypeStruct((16, 128), jnp.float32),
      scratch_shapes=[pltpu.VMEM((16, 128), jnp.float32)],
  )(batch3, h, rw1, rb1r, rw2p, rb2p)


def kernel(pos, x, batch, edge_index, node_table, Wq, Wk, Wv, We, Wo,
           ln1_s, ln1_b, W1, b1, W2, b2, ln2_s, ln2_b, RW1, rb1, RW2, rb2):
  f32 = jnp.float32
  row = edge_index[0].astype(jnp.int32)
  col = edge_index[1].astype(jnp.int32)

  pos128 = jnp.concatenate(
      [pos.astype(f32), jnp.zeros((_N, 125), f32)], axis=1)
  x2 = jnp.concatenate(
      [x.astype(jnp.int32), jnp.zeros((_NP - _N,), jnp.int32)]
  ).reshape(_NP, 1)
  batch3 = jnp.concatenate(
      [batch.astype(jnp.int32), jnp.full((_NP - _N,), 16, jnp.int32)]
  ).reshape(_NBLK, 1, 128)
  table_pad = jnp.concatenate(
      [node_table.astype(f32), jnp.zeros((128 - node_table.shape[0], _D), f32)],
      axis=0)
  offs = jnp.linspace(_CUTON, _CUTOFF, _D, dtype=f32).reshape(1, _D)
  znm = jnp.zeros((_NP, 128), f32)

  posr, posc = _sc_gather([pos128, pos128], [row, col])
  elen = _tc_elen(posr, posc)
  h = _tc_embed(x2, table_pad)
  for l in range(2):
    q, k, v = _tc_qkv(h, Wq[l], Wk[l], Wv[l])
    qr, kc, vc = _sc_gather([q, k, v], [row, col, col])
    msg, exd = _tc_edge(elen, qr, kc, vc, We[l], offs)
    nm, dnb = _sc_scatter(msg, exd, row, znm)
    h = _tc_update(h, nm, dnb,
                   Wo[l], ln1_s[l].reshape(1, _D), ln1_b[l].reshape(1, _D),
                   W1[l], b1[l].reshape(1, 2 * _D), W2[l],
                   b2[l].reshape(1, _D), ln2_s[l].reshape(1, _D),
                   ln2_b[l].reshape(1, _D))

  rw2p = jnp.concatenate([RW2.astype(f32), jnp.zeros((_D, 127), f32)], axis=1)
  rb2p = jnp.broadcast_to(rb2.astype(f32).reshape(1, 1), (1, 128))
  out = _tc_pool(batch3, h, RW1, rb1.reshape(1, _D), rw2p, rb2p)
  return out[:, 0]


# speedup vs baseline: 4.1726x; 1.4554x over previous
import functools

import jax
import jax.numpy as jnp
from jax import lax
from jax.experimental import pallas as pl
from jax.experimental.pallas import tpu as pltpu
from jax.experimental.pallas import tpu_sc as plsc

_N = 10000
_NP = 10112
_NBLK = _NP // 128
_E = 320000
_D = 128
_NW = 32
_EPW = _E // _NW
_GCH = 80
_GIT = _EPW // _GCH
_RPT = _NP // 16
_EB = 1280
_EG = _E // _EB
_CUTON = 3.0
_CUTOFF = 8.0
_WIDTH = (_CUTOFF - _CUTON) / (_D - 1)


def _mesh():
  return plsc.VectorSubcoreMesh(core_axis_name="c", subcore_axis_name="s")


def _sc_gather(tabs, idxs):
  n = len(tabs)
  widths = [t.shape[1] for t in tabs]

  scratch = []
  for t in range(n):
    for p in range(2):
      scratch.append(pltpu.VMEM((1, _GCH), jnp.int32))
      scratch.append(pltpu.VMEM((_GCH, widths[t]), jnp.float32))
  scratch += [pltpu.SemaphoreType.DMA, pltpu.SemaphoreType.DMA]

  @functools.partial(
      pl.kernel,
      mesh=_mesh(),
      out_type=[jax.ShapeDtypeStruct((_E, widths[t]), jnp.float32)
                for t in range(n)],
      scratch_types=scratch,
  )
  def gk(*refs):
    tab_r = refs[:n]
    idx_r = refs[n:2 * n]
    out_r = refs[2 * n:3 * n]
    sref = refs[3 * n:]
    ix = [[sref[4 * t + 2 * p] for p in range(2)] for t in range(n)]
    buf = [[sref[4 * t + 2 * p + 1] for p in range(2)] for t in range(n)]
    gsem = [sref[4 * n], sref[4 * n + 1]]
    c = lax.axis_index("c")
    s = lax.axis_index("s")
    base = (s * 2 + c) * _EPW

    def issue(j, p):
      off = base + j * _GCH
      for t in range(n):
        pltpu.sync_copy(idx_r[t].at[pl.ds(off, _GCH)], ix[t][p].at[0])
        pltpu.async_copy(tab_r[t].at[ix[t][p].at[0]], buf[t][p], gsem[p])

    def wait_writeout(j, p):
      off = base + j * _GCH
      for t in range(n):
        pltpu.make_async_copy(
            tab_r[t].at[ix[t][p].at[0]], buf[t][p], gsem[p]).wait()
      for t in range(n):
        pltpu.sync_copy(buf[t][p], out_r[t].at[pl.ds(off, _GCH)])

    issue(0, 0)

    def body(j, carry):
      for p in range(2):
        @pl.when((j & 1) == p)
        def _():
          @pl.when(j + 1 < _GIT)
          def _():
            issue(j + 1, 1 - p)
          wait_writeout(j, p)
      return carry

    lax.fori_loop(0, _GIT, body, 0)

  return gk(*tabs, *idxs)


def _sc_scatter(msg, exd, row, znm):
  epw = _E // 16
  its = epw // _GCH

  @functools.partial(
      pl.kernel,
      mesh=_mesh(),
      out_type=[
          jax.ShapeDtypeStruct((_NP, 128), jnp.float32),
          jax.ShapeDtypeStruct((_NP, 128), jnp.float32),
      ],
      scratch_types=[
          pltpu.VMEM((1, _GCH), jnp.int32),
          pltpu.VMEM((1, _GCH), jnp.int32),
          pltpu.VMEM((_GCH, 128), jnp.float32),
          pltpu.VMEM((_GCH, 128), jnp.float32),
          pltpu.VMEM_SHARED((_NP, 128), jnp.float32),
          pltpu.SemaphoreType.DMA,
          pltpu.SemaphoreType.DMA,
      ],
  )
  def sk(msg_h, exd_h, row_h, znm_h, nm_o, dn_o,
         ix0, ix1, mb0, mb1, acc, lsem0, lsem1):
    c = lax.axis_index("c")
    s = lax.axis_index("s")
    r0 = s * _RPT
    ix = [ix0, ix1]
    mb = [mb0, mb1]
    lsem = [lsem0, lsem1]
    pltpu.sync_copy(znm_h.at[pl.ds(r0, _RPT)], acc.at[pl.ds(r0, _RPT)])
    plsc.subcore_barrier()
    base = s * epw

    def body(src_h):
      def issue(j, p):
        off = base + j * _GCH
        pltpu.sync_copy(row_h.at[pl.ds(off, _GCH)], ix[p].at[0])
        pltpu.async_copy(src_h.at[pl.ds(off, _GCH)], mb[p], lsem[p])

      issue(0, 0)

      def step(j, carry):
        for p in range(2):
          @pl.when((j & 1) == p)
          def _():
            @pl.when(j + 1 < its)
            def _():
              issue(j + 1, 1 - p)
            off = base + j * _GCH
            pltpu.make_async_copy(
                src_h.at[pl.ds(off, _GCH)], mb[p], lsem[p]).wait()
            pltpu.sync_copy(mb[p], acc.at[ix[p].at[0]], add=True)
        return carry

      lax.fori_loop(0, its, step, 0)

    @pl.when(c == 0)
    def _():
      body(msg_h)

    @pl.when(c == 1)
    def _():
      body(exd_h)

    plsc.subcore_barrier()

    @pl.when(c == 0)
    def _():
      pltpu.sync_copy(acc.at[pl.ds(r0, _RPT)], nm_o.at[pl.ds(r0, _RPT)])

    @pl.when(c == 1)
    def _():
      pltpu.sync_copy(acc.at[pl.ds(r0, _RPT)], dn_o.at[pl.ds(r0, _RPT)])

  return sk(msg, exd, row, znm)


def _tc_embed(x2, table_pad):
  def body(x_ref, t_ref, o_ref):
    xi = x_ref[...]
    oh = (xi == lax.broadcasted_iota(jnp.int32, (128, 128), 1)
          ).astype(jnp.float32)
    o_ref[...] = jnp.dot(oh, t_ref[...], preferred_element_type=jnp.float32, precision=lax.Precision.HIGHEST)

  return pl.pallas_call(
      body,
      grid=(_NBLK,),
      in_specs=[
          pl.BlockSpec((128, 1), lambda i: (i, 0)),
          pl.BlockSpec((128, 128), lambda i: (0, 0)),
      ],
      out_specs=pl.BlockSpec((128, 128), lambda i: (i, 0)),
      out_shape=jax.ShapeDtypeStruct((_NP, 128), jnp.float32),
  )(x2, table_pad)


def _tc_qkv(h, wq, wk, wv):
  def body(h_ref, q_w, k_w, v_w, q_o, kv_o):
    hb = h_ref[...]
    q_o[...] = jnp.dot(hb, q_w[...], preferred_element_type=jnp.float32, precision=lax.Precision.HIGHEST)
    kv_o[...] = jnp.concatenate([
        jnp.dot(hb, k_w[...], preferred_element_type=jnp.float32, precision=lax.Precision.HIGHEST),
        jnp.dot(hb, v_w[...], preferred_element_type=jnp.float32, precision=lax.Precision.HIGHEST),
    ], axis=1)

  wspec = pl.BlockSpec((128, 128), lambda i: (0, 0))
  hspec = pl.BlockSpec((128, 128), lambda i: (i, 0))
  return pl.pallas_call(
      body,
      grid=(_NBLK,),
      in_specs=[hspec, wspec, wspec, wspec],
      out_specs=[hspec, pl.BlockSpec((128, 256), lambda i: (i, 0))],
      out_shape=[
          jax.ShapeDtypeStruct((_NP, 128), jnp.float32),
          jax.ShapeDtypeStruct((_NP, 256), jnp.float32),
      ],
  )(h, wq, wk, wv)


def _tc_elen(posr, posc):
  def body(pr, pc, el_o):
    d = pr[...] - pc[...]
    d2 = jnp.sum(d * d, axis=1, keepdims=True)
    el_o[...] = jnp.sqrt(d2)

  espec = pl.BlockSpec((_EB, 128), lambda i: (i, 0))
  return pl.pallas_call(
      body,
      grid=(_EG,),
      in_specs=[espec, espec],
      out_specs=pl.BlockSpec((_EB, 1), lambda i: (i, 0)),
      out_shape=jax.ShapeDtypeStruct((_E, 1), jnp.float32),
  )(posr, posc)


def _tc_edge(elen, qr, kv_g, we_l, offs):
  def body(el_r, q_r, kv_r, w_r, of_r, msg_o, exd_o):
    el = el_r[...]
    a = (el - of_r[...]) * (1.0 / _WIDTH)
    attr = jnp.exp(-0.5 * a * a)
    e = jnp.dot(attr, w_r[...], preferred_element_type=jnp.float32, precision=lax.Precision.HIGHEST)
    kvb = kv_r[...]
    prod = q_r[...] * kvb[:, :128] * e
    r16 = lax.broadcasted_iota(jnp.int32, (128, 16), 0) // 16
    c16 = lax.broadcasted_iota(jnp.int32, (128, 16), 1)
    seg = (r16 == c16).astype(jnp.float32)
    s16 = jnp.dot(prod, seg, preferred_element_type=jnp.float32, precision=lax.Precision.HIGHEST) * 0.25
    ex = jnp.exp(s16)
    rt = lax.broadcasted_iota(jnp.int32, (16, 128), 0)
    ct = lax.broadcasted_iota(jnp.int32, (16, 128), 1) // 16
    segt = (rt == ct).astype(jnp.float32)
    exd = jnp.dot(ex, segt, preferred_element_type=jnp.float32, precision=lax.Precision.HIGHEST)
    exd_o[...] = exd
    msg_o[...] = exd * kvb[:, 128:]

  espec = pl.BlockSpec((_EB, 128), lambda i: (i, 0))
  return pl.pallas_call(
      body,
      grid=(_EG,),
      in_specs=[
          pl.BlockSpec((_EB, 1), lambda i: (i, 0)),
          espec,
          pl.BlockSpec((_EB, 256), lambda i: (i, 0)),
          pl.BlockSpec((128, 128), lambda i: (0, 0)),
          pl.BlockSpec((1, 128), lambda i: (0, 0)),
      ],
      out_specs=[espec, espec],
      out_shape=[
          jax.ShapeDtypeStruct((_E, 128), jnp.float32),
          jax.ShapeDtypeStruct((_E, 128), jnp.float32),
      ],
  )(elen, qr, kv_g, we_l, offs)


def _layernorm(t, s_row, b_row):
  mu = jnp.mean(t, axis=-1, keepdims=True)
  var = jnp.mean((t - mu) * (t - mu), axis=-1, keepdims=True)
  return (t - mu) / jnp.sqrt(var + 1e-5) * s_row + b_row


def _tc_update(h, nm, dnb, wo, l1s, l1b, w1, b1, w2, b2, l2s, l2b):
  def body(h_ref, n_r, d_r, wo_r, l1s_r, l1b_r, w1_r, b1_r,
           w2_r, b2_r, l2s_r, l2b_r, o_ref):
    agg = n_r[...] / (d_r[...] + 1e-9)
    t1 = _layernorm(
        h_ref[...] + jnp.dot(agg, wo_r[...],
                             preferred_element_type=jnp.float32, precision=lax.Precision.HIGHEST),
        l1s_r[...], l1b_r[...])
    ff = jnp.dot(
        jnp.maximum(
            jnp.dot(t1, w1_r[...], preferred_element_type=jnp.float32, precision=lax.Precision.HIGHEST)
            + b1_r[...], 0.0),
        w2_r[...], preferred_element_type=jnp.float32, precision=lax.Precision.HIGHEST) + b2_r[...]
    o_ref[...] = _layernorm(t1 + ff, l2s_r[...], l2b_r[...])

  hspec = pl.BlockSpec((128, 128), lambda i: (i, 0))
  return pl.pallas_call(
      body,
      grid=(_NBLK,),
      in_specs=[
          hspec,
          hspec,
          hspec,
          pl.BlockSpec((128, 128), lambda i: (0, 0)),
          pl.BlockSpec((1, 128), lambda i: (0, 0)),
          pl.BlockSpec((1, 128), lambda i: (0, 0)),
          pl.BlockSpec((128, 256), lambda i: (0, 0)),
          pl.BlockSpec((1, 256), lambda i: (0, 0)),
          pl.BlockSpec((256, 128), lambda i: (0, 0)),
          pl.BlockSpec((1, 128), lambda i: (0, 0)),
          pl.BlockSpec((1, 128), lambda i: (0, 0)),
          pl.BlockSpec((1, 128), lambda i: (0, 0)),
      ],
      out_specs=hspec,
      out_shape=jax.ShapeDtypeStruct((_NP, 128), jnp.float32),
  )(h, nm, dnb, wo, l1s, l1b, w1, b1, w2, b2, l2s, l2b)


def _tc_pool(batch3, h, rw1, rb1r, rw2p, rb2p):
  def body(b_ref, h_ref, w1_r, b1_r, w2_r, b2_r, o_ref, acc):
    i = pl.program_id(0)

    @pl.when(i == 0)
    def _():
      acc[...] = jnp.zeros_like(acc)

    bv = b_ref[0]
    oh = (jnp.broadcast_to(bv, (16, 128))
          == lax.broadcasted_iota(jnp.int32, (16, 128), 0)
          ).astype(jnp.float32)
    acc[...] += jnp.dot(oh, h_ref[...], preferred_element_type=jnp.float32, precision=lax.Precision.HIGHEST)

    @pl.when(i == _NBLK - 1)
    def _():
      z = jnp.maximum(
          jnp.dot(acc[...], w1_r[...], preferred_element_type=jnp.float32, precision=lax.Precision.HIGHEST)
          + b1_r[...], 0.0)
      o_ref[...] = jnp.dot(
          z, w2_r[...], preferred_element_type=jnp.float32, precision=lax.Precision.HIGHEST) + b2_r[...]

  return pl.pallas_call(
      body,
      grid=(_NBLK,),
      in_specs=[
          pl.BlockSpec((1, 1, 128), lambda i: (i, 0, 0)),
          pl.BlockSpec((128, 128), lambda i: (i, 0)),
          pl.BlockSpec((128, 128), lambda i: (0, 0)),
          pl.BlockSpec((1, 128), lambda i: (0, 0)),
          pl.BlockSpec((128, 128), lambda i: (0, 0)),
          pl.BlockSpec((1, 128), lambda i: (0, 0)),
      ],
      out_specs=pl.BlockSpec((16, 128), lambda i: (0, 0)),
      out_shape=jax.ShapeDtypeStruct((16, 128), jnp.float32),
      scratch_shapes=[pltpu.VMEM((16, 128), jnp.float32)],
  )(batch3, h, rw1, rb1r, rw2p, rb2p)


def kernel(pos, x, batch, edge_index, node_table, Wq, Wk, Wv, We, Wo,
           ln1_s, ln1_b, W1, b1, W2, b2, ln2_s, ln2_b, RW1, rb1, RW2, rb2):
  f32 = jnp.float32
  row = edge_index[0].astype(jnp.int32)
  col = edge_index[1].astype(jnp.int32)

  pos128 = jnp.concatenate(
      [pos.astype(f32), jnp.zeros((_N, 125), f32)], axis=1)
  x2 = jnp.concatenate(
      [x.astype(jnp.int32), jnp.zeros((_NP - _N,), jnp.int32)]
  ).reshape(_NP, 1)
  batch3 = jnp.concatenate(
      [batch.astype(jnp.int32), jnp.full((_NP - _N,), 16, jnp.int32)]
  ).reshape(_NBLK, 1, 128)
  table_pad = jnp.concatenate(
      [node_table.astype(f32), jnp.zeros((128 - node_table.shape[0], _D), f32)],
      axis=0)
  offs = jnp.linspace(_CUTON, _CUTOFF, _D, dtype=f32).reshape(1, _D)
  znm = jnp.zeros((_NP, 128), f32)

  posr, posc = _sc_gather([pos128, pos128], [row, col])
  elen = _tc_elen(posr, posc)
  h = _tc_embed(x2, table_pad)
  for l in range(2):
    q, kv = _tc_qkv(h, Wq[l], Wk[l], Wv[l])
    qr, kvg = _sc_gather([q, kv], [row, col])
    msg, exd = _tc_edge(elen, qr, kvg, We[l], offs)
    nm, dnb = _sc_scatter(msg, exd, row, znm)
    h = _tc_update(h, nm, dnb,
                   Wo[l], ln1_s[l].reshape(1, _D), ln1_b[l].reshape(1, _D),
                   W1[l], b1[l].reshape(1, 2 * _D), W2[l],
                   b2[l].reshape(1, _D), ln2_s[l].reshape(1, _D),
                   ln2_b[l].reshape(1, _D))

  rw2p = jnp.concatenate([RW2.astype(f32), jnp.zeros((_D, 127), f32)], axis=1)
  rb2p = jnp.broadcast_to(rb2.astype(f32).reshape(1, 1), (1, 128))
  out = _tc_pool(batch3, h, RW1, rb1.reshape(1, _D), rw2p, rb2p)
  return out[:, 0]


# speedup vs baseline: 4.5380x; 1.0876x over previous
import functools

import jax
import jax.numpy as jnp
from jax import lax
from jax.experimental import pallas as pl
from jax.experimental.pallas import tpu as pltpu
from jax.experimental.pallas import tpu_sc as plsc

_N = 10000
_NP = 10112
_NBLK = _NP // 128
_E = 320000
_D = 128
_NW = 32
_EPW = _E // _NW
_GCH = 80
_GIT = _EPW // _GCH
_RPT = _NP // 16
_EB = 1280
_EG = _E // _EB
_CUTON = 3.0
_CUTOFF = 8.0
_WIDTH = (_CUTOFF - _CUTON) / (_D - 1)


def _mesh():
  return plsc.VectorSubcoreMesh(core_axis_name="c", subcore_axis_name="s")


def _sc_gather(tabs, idxs, gch=_GCH):
  n = len(tabs)
  widths = [t.shape[1] for t in tabs]
  ecnt = idxs[0].shape[0]
  epw = ecnt // _NW
  git = epw // gch

  scratch = []
  for t in range(n):
    for p in range(2):
      scratch.append(pltpu.VMEM((1, gch), jnp.int32))
      scratch.append(pltpu.VMEM((gch, widths[t]), jnp.float32))
  scratch += [pltpu.SemaphoreType.DMA, pltpu.SemaphoreType.DMA]

  @functools.partial(
      pl.kernel,
      mesh=_mesh(),
      out_type=[jax.ShapeDtypeStruct((ecnt, widths[t]), jnp.float32)
                for t in range(n)],
      scratch_types=scratch,
  )
  def gk(*refs):
    tab_r = refs[:n]
    idx_r = refs[n:2 * n]
    out_r = refs[2 * n:3 * n]
    sref = refs[3 * n:]
    ix = [[sref[4 * t + 2 * p] for p in range(2)] for t in range(n)]
    buf = [[sref[4 * t + 2 * p + 1] for p in range(2)] for t in range(n)]
    gsem = [sref[4 * n], sref[4 * n + 1]]
    c = lax.axis_index("c")
    s = lax.axis_index("s")
    base = (s * 2 + c) * epw

    def issue(j, p):
      off = base + j * gch
      for t in range(n):
        pltpu.sync_copy(idx_r[t].at[pl.ds(off, gch)], ix[t][p].at[0])
        pltpu.async_copy(tab_r[t].at[ix[t][p].at[0]], buf[t][p], gsem[p])

    def wait_writeout(j, p):
      off = base + j * gch
      for t in range(n):
        pltpu.make_async_copy(
            tab_r[t].at[ix[t][p].at[0]], buf[t][p], gsem[p]).wait()
      for t in range(n):
        pltpu.sync_copy(buf[t][p], out_r[t].at[pl.ds(off, gch)])

    issue(0, 0)

    def body(j, carry):
      for p in range(2):
        @pl.when((j & 1) == p)
        def _():
          @pl.when(j + 1 < git)
          def _():
            issue(j + 1, 1 - p)
          wait_writeout(j, p)
      return carry

    lax.fori_loop(0, git, body, 0)

  return gk(*tabs, *idxs)


def _sc_scatter(msg, exd, row, znm):
  ecnt = msg.shape[0]
  epw = ecnt // 16
  its = epw // _GCH

  @functools.partial(
      pl.kernel,
      mesh=_mesh(),
      out_type=[
          jax.ShapeDtypeStruct((_NP, 128), jnp.float32),
          jax.ShapeDtypeStruct((_NP, 128), jnp.float32),
      ],
      scratch_types=[
          pltpu.VMEM((1, _GCH), jnp.int32),
          pltpu.VMEM((1, _GCH), jnp.int32),
          pltpu.VMEM((_GCH, 128), jnp.float32),
          pltpu.VMEM((_GCH, 128), jnp.float32),
          pltpu.VMEM_SHARED((_NP, 128), jnp.float32),
          pltpu.SemaphoreType.DMA,
          pltpu.SemaphoreType.DMA,
      ],
  )
  def sk(msg_h, exd_h, row_h, znm_h, nm_o, dn_o,
         ix0, ix1, mb0, mb1, acc, lsem0, lsem1):
    c = lax.axis_index("c")
    s = lax.axis_index("s")
    r0 = s * _RPT
    ix = [ix0, ix1]
    mb = [mb0, mb1]
    lsem = [lsem0, lsem1]
    pltpu.sync_copy(znm_h.at[pl.ds(r0, _RPT)], acc.at[pl.ds(r0, _RPT)])
    plsc.subcore_barrier()
    base = s * epw

    def body(src_h):
      def issue(j, p):
        off = base + j * _GCH
        pltpu.sync_copy(row_h.at[pl.ds(off, _GCH)], ix[p].at[0])
        pltpu.async_copy(src_h.at[pl.ds(off, _GCH)], mb[p], lsem[p])

      issue(0, 0)

      def step(j, carry):
        for p in range(2):
          @pl.when((j & 1) == p)
          def _():
            @pl.when(j + 1 < its)
            def _():
              issue(j + 1, 1 - p)
            off = base + j * _GCH
            pltpu.make_async_copy(
                src_h.at[pl.ds(off, _GCH)], mb[p], lsem[p]).wait()
            pltpu.sync_copy(mb[p], acc.at[ix[p].at[0]], add=True)
        return carry

      lax.fori_loop(0, its, step, 0)

    @pl.when(c == 0)
    def _():
      body(msg_h)

    @pl.when(c == 1)
    def _():
      body(exd_h)

    plsc.subcore_barrier()

    @pl.when(c == 0)
    def _():
      pltpu.sync_copy(acc.at[pl.ds(r0, _RPT)], nm_o.at[pl.ds(r0, _RPT)])

    @pl.when(c == 1)
    def _():
      pltpu.sync_copy(acc.at[pl.ds(r0, _RPT)], dn_o.at[pl.ds(r0, _RPT)])

  return sk(msg, exd, row, znm)


def _qkv_of(hb, q_w, k_w, v_w):
  q = jnp.dot(hb, q_w[...], preferred_element_type=jnp.float32, precision=lax.Precision.HIGHEST)
  kv = jnp.concatenate([
      jnp.dot(hb, k_w[...], preferred_element_type=jnp.float32, precision=lax.Precision.HIGHEST),
      jnp.dot(hb, v_w[...], preferred_element_type=jnp.float32, precision=lax.Precision.HIGHEST),
  ], axis=1)
  return q, kv


def _tc_embed_qkv(x2, table_pad, wq, wk, wv):
  def body(x_ref, t_ref, q_w, k_w, v_w, h_o, q_o, kv_o):
    xi = x_ref[...]
    oh = (xi == lax.broadcasted_iota(jnp.int32, (128, 128), 1)
          ).astype(jnp.float32)
    hb = jnp.dot(oh, t_ref[...], preferred_element_type=jnp.float32, precision=lax.Precision.HIGHEST)
    h_o[...] = hb
    q, kv = _qkv_of(hb, q_w, k_w, v_w)
    q_o[...] = q
    kv_o[...] = kv

  wspec = pl.BlockSpec((128, 128), lambda i: (0, 0))
  hspec = pl.BlockSpec((128, 128), lambda i: (i, 0))
  return pl.pallas_call(
      body,
      grid=(_NBLK,),
      in_specs=[pl.BlockSpec((128, 1), lambda i: (i, 0)),
                wspec, wspec, wspec, wspec],
      out_specs=[hspec, hspec, pl.BlockSpec((128, 256), lambda i: (i, 0))],
      out_shape=[
          jax.ShapeDtypeStruct((_NP, 128), jnp.float32),
          jax.ShapeDtypeStruct((_NP, 128), jnp.float32),
          jax.ShapeDtypeStruct((_NP, 256), jnp.float32),
      ],
  )(x2, table_pad, wq, wk, wv)


def _tc_elen(posr, posc):
  def body(pr, pc, el_o):
    d = pr[...] - pc[...]
    d2 = jnp.sum(d * d, axis=1, keepdims=True)
    el_o[...] = jnp.sqrt(d2)

  espec = pl.BlockSpec((_EB, 128), lambda i: (i, 0))
  return pl.pallas_call(
      body,
      grid=(_EG,),
      in_specs=[espec, espec],
      out_specs=pl.BlockSpec((_EB, 1), lambda i: (i, 0)),
      out_shape=jax.ShapeDtypeStruct((_E, 1), jnp.float32),
  )(posr, posc)


def _tc_edge(elen, qr, kv_g, we_l, offs):
  def body(el_r, q_r, kv_r, w_r, of_r, msg_o, exd_o):
    el = el_r[...]
    a = (el - of_r[...]) * (1.0 / _WIDTH)
    attr = jnp.exp(-0.5 * a * a)
    e = jnp.dot(attr, w_r[...], preferred_element_type=jnp.float32, precision=lax.Precision.HIGHEST)
    kvb = kv_r[...]
    prod = q_r[...] * kvb[:, :128] * e
    r16 = lax.broadcasted_iota(jnp.int32, (128, 16), 0) // 16
    c16 = lax.broadcasted_iota(jnp.int32, (128, 16), 1)
    seg = (r16 == c16).astype(jnp.float32)
    s16 = jnp.dot(prod, seg, preferred_element_type=jnp.float32, precision=lax.Precision.HIGHEST) * 0.25
    ex = jnp.exp(s16)
    rt = lax.broadcasted_iota(jnp.int32, (16, 128), 0)
    ct = lax.broadcasted_iota(jnp.int32, (16, 128), 1) // 16
    segt = (rt == ct).astype(jnp.float32)
    exd = jnp.dot(ex, segt, preferred_element_type=jnp.float32, precision=lax.Precision.HIGHEST)
    exd_o[...] = exd
    msg_o[...] = exd * kvb[:, 128:]

  ecnt = elen.shape[0]
  espec = pl.BlockSpec((_EB, 128), lambda i: (i, 0))
  return pl.pallas_call(
      body,
      grid=(ecnt // _EB,),
      in_specs=[
          pl.BlockSpec((_EB, 1), lambda i: (i, 0)),
          espec,
          pl.BlockSpec((_EB, 256), lambda i: (i, 0)),
          pl.BlockSpec((128, 128), lambda i: (0, 0)),
          pl.BlockSpec((1, 128), lambda i: (0, 0)),
      ],
      out_specs=[espec, espec],
      out_shape=[
          jax.ShapeDtypeStruct((ecnt, 128), jnp.float32),
          jax.ShapeDtypeStruct((ecnt, 128), jnp.float32),
      ],
  )(elen, qr, kv_g, we_l, offs)


def _layernorm(t, s_row, b_row):
  mu = jnp.mean(t, axis=-1, keepdims=True)
  var = jnp.mean((t - mu) * (t - mu), axis=-1, keepdims=True)
  return (t - mu) / jnp.sqrt(var + 1e-5) * s_row + b_row


def _tc_update(h, nm, nm2, dnb, dnb2, wo, l1s, l1b, w1, b1, w2, b2, l2s, l2b):
  def body(h_ref, n_r, n2_r, d_r, d2_r, wo_r, l1s_r, l1b_r, w1_r, b1_r,
           w2_r, b2_r, l2s_r, l2b_r, o_ref):
    agg = (n_r[...] + n2_r[...]) / (d_r[...] + d2_r[...] + 1e-9)
    t1 = _layernorm(
        h_ref[...] + jnp.dot(agg, wo_r[...],
                             preferred_element_type=jnp.float32, precision=lax.Precision.HIGHEST),
        l1s_r[...], l1b_r[...])
    ff = jnp.dot(
        jnp.maximum(
            jnp.dot(t1, w1_r[...], preferred_element_type=jnp.float32, precision=lax.Precision.HIGHEST)
            + b1_r[...], 0.0),
        w2_r[...], preferred_element_type=jnp.float32, precision=lax.Precision.HIGHEST) + b2_r[...]
    o_ref[...] = _layernorm(t1 + ff, l2s_r[...], l2b_r[...])

  hspec = pl.BlockSpec((128, 128), lambda i: (i, 0))
  return pl.pallas_call(
      body,
      grid=(_NBLK,),
      in_specs=[
          hspec,
          hspec,
          hspec,
          hspec,
          hspec,
          pl.BlockSpec((128, 128), lambda i: (0, 0)),
          pl.BlockSpec((1, 128), lambda i: (0, 0)),
          pl.BlockSpec((1, 128), lambda i: (0, 0)),
          pl.BlockSpec((128, 256), lambda i: (0, 0)),
          pl.BlockSpec((1, 256), lambda i: (0, 0)),
          pl.BlockSpec((256, 128), lambda i: (0, 0)),
          pl.BlockSpec((1, 128), lambda i: (0, 0)),
          pl.BlockSpec((1, 128), lambda i: (0, 0)),
          pl.BlockSpec((1, 128), lambda i: (0, 0)),
      ],
      out_specs=hspec,
      out_shape=jax.ShapeDtypeStruct((_NP, 128), jnp.float32),
  )(h, nm, nm2, dnb, dnb2, wo, l1s, l1b, w1, b1, w2, b2, l2s, l2b)


def _tc_update_qkv(h, nm, nm2, dnb, dnb2, wo, l1s, l1b, w1, b1, w2, b2,
                   l2s, l2b, wq, wk, wv):
  def body(h_ref, n_r, n2_r, d_r, d2_r, wo_r, l1s_r, l1b_r, w1_r, b1_r,
           w2_r, b2_r, l2s_r, l2b_r, q_w, k_w, v_w, h_o, q_o, kv_o):
    agg = (n_r[...] + n2_r[...]) / (d_r[...] + d2_r[...] + 1e-9)
    t1 = _layernorm(
        h_ref[...] + jnp.dot(agg, wo_r[...],
                             preferred_element_type=jnp.float32, precision=lax.Precision.HIGHEST),
        l1s_r[...], l1b_r[...])
    ff = jnp.dot(
        jnp.maximum(
            jnp.dot(t1, w1_r[...], preferred_element_type=jnp.float32, precision=lax.Precision.HIGHEST)
            + b1_r[...], 0.0),
        w2_r[...], preferred_element_type=jnp.float32, precision=lax.Precision.HIGHEST) + b2_r[...]
    h2 = _layernorm(t1 + ff, l2s_r[...], l2b_r[...])
    h_o[...] = h2
    q, kv = _qkv_of(h2, q_w, k_w, v_w)
    q_o[...] = q
    kv_o[...] = kv

  hspec = pl.BlockSpec((128, 128), lambda i: (i, 0))
  wspec = pl.BlockSpec((128, 128), lambda i: (0, 0))
  rspec = pl.BlockSpec((1, 128), lambda i: (0, 0))
  return pl.pallas_call(
      body,
      grid=(_NBLK,),
      in_specs=[
          hspec, hspec, hspec, hspec, hspec,
          wspec, rspec, rspec,
          pl.BlockSpec((128, 256), lambda i: (0, 0)),
          pl.BlockSpec((1, 256), lambda i: (0, 0)),
          pl.BlockSpec((256, 128), lambda i: (0, 0)),
          rspec, rspec, rspec,
          wspec, wspec, wspec,
      ],
      out_specs=[hspec, hspec, pl.BlockSpec((128, 256), lambda i: (i, 0))],
      out_shape=[
          jax.ShapeDtypeStruct((_NP, 128), jnp.float32),
          jax.ShapeDtypeStruct((_NP, 128), jnp.float32),
          jax.ShapeDtypeStruct((_NP, 256), jnp.float32),
      ],
  )(h, nm, nm2, dnb, dnb2, wo, l1s, l1b, w1, b1, w2, b2, l2s, l2b,
    wq, wk, wv)


def _tc_pool(batch3, h, rw1, rb1r, rw2p, rb2p):
  def body(b_ref, h_ref, w1_r, b1_r, w2_r, b2_r, o_ref, acc):
    i = pl.program_id(0)

    @pl.when(i == 0)
    def _():
      acc[...] = jnp.zeros_like(acc)

    bv = b_ref[0]
    oh = (jnp.broadcast_to(bv, (16, 128))
          == lax.broadcasted_iota(jnp.int32, (16, 128), 0)
          ).astype(jnp.float32)
    acc[...] += jnp.dot(oh, h_ref[...], preferred_element_type=jnp.float32, precision=lax.Precision.HIGHEST)

    @pl.when(i == _NBLK - 1)
    def _():
      z = jnp.maximum(
          jnp.dot(acc[...], w1_r[...], preferred_element_type=jnp.float32, precision=lax.Precision.HIGHEST)
          + b1_r[...], 0.0)
      o_ref[...] = jnp.dot(
          z, w2_r[...], preferred_element_type=jnp.float32, precision=lax.Precision.HIGHEST) + b2_r[...]

  return pl.pallas_call(
      body,
      grid=(_NBLK,),
      in_specs=[
          pl.BlockSpec((1, 1, 128), lambda i: (i, 0, 0)),
          pl.BlockSpec((128, 128), lambda i: (i, 0)),
          pl.BlockSpec((128, 128), lambda i: (0, 0)),
          pl.BlockSpec((1, 128), lambda i: (0, 0)),
          pl.BlockSpec((128, 128), lambda i: (0, 0)),
          pl.BlockSpec((1, 128), lambda i: (0, 0)),
      ],
      out_specs=pl.BlockSpec((16, 128), lambda i: (0, 0)),
      out_shape=jax.ShapeDtypeStruct((16, 128), jnp.float32),
      scratch_shapes=[pltpu.VMEM((16, 128), jnp.float32)],
  )(batch3, h, rw1, rb1r, rw2p, rb2p)


def kernel(pos, x, batch, edge_index, node_table, Wq, Wk, Wv, We, Wo,
           ln1_s, ln1_b, W1, b1, W2, b2, ln2_s, ln2_b, RW1, rb1, RW2, rb2):
  f32 = jnp.float32
  row = edge_index[0].astype(jnp.int32)
  col = edge_index[1].astype(jnp.int32)

  pos128 = jnp.concatenate(
      [pos.astype(f32), jnp.zeros((_N, 125), f32)], axis=1)
  x2 = jnp.concatenate(
      [x.astype(jnp.int32), jnp.zeros((_NP - _N,), jnp.int32)]
  ).reshape(_NP, 1)
  batch3 = jnp.concatenate(
      [batch.astype(jnp.int32), jnp.full((_NP - _N,), 16, jnp.int32)]
  ).reshape(_NBLK, 1, 128)
  table_pad = jnp.concatenate(
      [node_table.astype(f32), jnp.zeros((128 - node_table.shape[0], _D), f32)],
      axis=0)
  offs = jnp.linspace(_CUTON, _CUTOFF, _D, dtype=f32).reshape(1, _D)
  znm = jnp.zeros((_NP, 128), f32)

  posr, posc = _sc_gather([pos128, pos128], [row, col])
  elen = _tc_elen(posr, posc)
  h, q, kv = _tc_embed_qkv(x2, table_pad, Wq[0], Wk[0], Wv[0])

  e2 = _E // 2
  rowA, rowB = row[:e2], row[e2:]
  colA, colB = col[:e2], col[e2:]
  elenA, elenB = elen[:e2], elen[e2:]

  def edge_phase(q, kv, we_l):
    qrA, kvA = _sc_gather([q, kv], [rowA, colA], gch=40)
    msgA, exdA = _tc_edge(elenA, qrA, kvA, we_l, offs)
    qrB, kvB = _sc_gather([q, kv], [rowB, colB], gch=40)
    msgB, exdB = _tc_edge(elenB, qrB, kvB, we_l, offs)
    nmA, dnA = _sc_scatter(msgA, exdA, rowA, znm)
    nmB, dnB = _sc_scatter(msgB, exdB, rowB, znm)
    return nmA, nmB, dnA, dnB

  nmA, nmB, dnA, dnB = edge_phase(q, kv, We[0])
  h, q, kv = _tc_update_qkv(
      h, nmA, nmB, dnA, dnB, Wo[0],
      ln1_s[0].reshape(1, _D), ln1_b[0].reshape(1, _D),
      W1[0], b1[0].reshape(1, 2 * _D), W2[0], b2[0].reshape(1, _D),
      ln2_s[0].reshape(1, _D), ln2_b[0].reshape(1, _D),
      Wq[1], Wk[1], Wv[1])

  nmA, nmB, dnA, dnB = edge_phase(q, kv, We[1])
  h = _tc_update(h, nmA, nmB, dnA, dnB,
                 Wo[1], ln1_s[1].reshape(1, _D), ln1_b[1].reshape(1, _D),
                 W1[1], b1[1].reshape(1, 2 * _D), W2[1],
                 b2[1].reshape(1, _D), ln2_s[1].reshape(1, _D),
                 ln2_b[1].reshape(1, _D))

  rw2p = jnp.concatenate([RW2.astype(f32), jnp.zeros((_D, 127), f32)], axis=1)
  rb2p = jnp.broadcast_to(rb2.astype(f32).reshape(1, 1), (1, 128))
  out = _tc_pool(batch3, h, RW1, rb1.reshape(1, _D), rw2p, rb2p)
  return out[:, 0]


# speedup vs baseline: 4.9221x; 1.0846x over previous
import functools

import jax
import jax.numpy as jnp
from jax import lax
from jax.experimental import pallas as pl
from jax.experimental.pallas import tpu as pltpu
from jax.experimental.pallas import tpu_sc as plsc

_N = 10000
_NP = 10112
_NBLK = _NP // 128
_E = 320000
_D = 128
_NW = 32
_EPW = _E // _NW
_GCH = 80
_GIT = _EPW // _GCH
_RPT = _NP // 16
_EB = 1280
_EG = _E // _EB
_CUTON = 3.0
_CUTOFF = 8.0
_WIDTH = (_CUTOFF - _CUTON) / (_D - 1)


def _mesh():
  return plsc.VectorSubcoreMesh(core_axis_name="c", subcore_axis_name="s")


def _sc_gather(tabs, idxs, gch=_GCH):
  n = len(tabs)
  widths = [t.shape[1] for t in tabs]
  ecnt = idxs[0].shape[0]
  epw = ecnt // _NW
  git = epw // gch

  scratch = []
  for t in range(n):
    for p in range(2):
      scratch.append(pltpu.VMEM((1, gch), jnp.int32))
      scratch.append(pltpu.VMEM((gch, widths[t]), jnp.float32))
  scratch += [pltpu.SemaphoreType.DMA, pltpu.SemaphoreType.DMA]

  @functools.partial(
      pl.kernel,
      mesh=_mesh(),
      out_type=[jax.ShapeDtypeStruct((ecnt, widths[t]), jnp.float32)
                for t in range(n)],
      scratch_types=scratch,
  )
  def gk(*refs):
    tab_r = refs[:n]
    idx_r = refs[n:2 * n]
    out_r = refs[2 * n:3 * n]
    sref = refs[3 * n:]
    ix = [[sref[4 * t + 2 * p] for p in range(2)] for t in range(n)]
    buf = [[sref[4 * t + 2 * p + 1] for p in range(2)] for t in range(n)]
    gsem = [sref[4 * n], sref[4 * n + 1]]
    c = lax.axis_index("c")
    s = lax.axis_index("s")
    base = (s * 2 + c) * epw

    def issue(j, p):
      off = base + j * gch
      for t in range(n):
        pltpu.sync_copy(idx_r[t].at[pl.ds(off, gch)], ix[t][p].at[0])
        pltpu.async_copy(tab_r[t].at[ix[t][p].at[0]], buf[t][p], gsem[p])

    def wait_writeout(j, p):
      off = base + j * gch
      for t in range(n):
        pltpu.make_async_copy(
            tab_r[t].at[ix[t][p].at[0]], buf[t][p], gsem[p]).wait()
      for t in range(n):
        pltpu.sync_copy(buf[t][p], out_r[t].at[pl.ds(off, gch)])

    issue(0, 0)

    def body(j, carry):
      for p in range(2):
        @pl.when((j & 1) == p)
        def _():
          @pl.when(j + 1 < git)
          def _():
            issue(j + 1, 1 - p)
          wait_writeout(j, p)
      return carry

    lax.fori_loop(0, git, body, 0)

  return gk(*tabs, *idxs)


def _sc_scatter(msg, exd, row, znm):
  ecnt = msg.shape[0]
  epw = ecnt // 16
  its = epw // _GCH

  @functools.partial(
      pl.kernel,
      mesh=_mesh(),
      out_type=[
          jax.ShapeDtypeStruct((_NP, 128), jnp.float32),
          jax.ShapeDtypeStruct((_NP, 128), jnp.float32),
      ],
      scratch_types=[
          pltpu.VMEM((1, _GCH), jnp.int32),
          pltpu.VMEM((1, _GCH), jnp.int32),
          pltpu.VMEM((_GCH, 128), jnp.float32),
          pltpu.VMEM((_GCH, 128), jnp.float32),
          pltpu.VMEM_SHARED((_NP, 128), jnp.float32),
          pltpu.SemaphoreType.DMA,
          pltpu.SemaphoreType.DMA,
      ],
  )
  def sk(msg_h, exd_h, row_h, znm_h, nm_o, dn_o,
         ix0, ix1, mb0, mb1, acc, lsem0, lsem1):
    c = lax.axis_index("c")
    s = lax.axis_index("s")
    r0 = s * _RPT
    ix = [ix0, ix1]
    mb = [mb0, mb1]
    lsem = [lsem0, lsem1]
    pltpu.sync_copy(znm_h.at[pl.ds(r0, _RPT)], acc.at[pl.ds(r0, _RPT)])
    plsc.subcore_barrier()
    base = s * epw

    def body(src_h):
      def issue(j, p):
        off = base + j * _GCH
        pltpu.sync_copy(row_h.at[pl.ds(off, _GCH)], ix[p].at[0])
        pltpu.async_copy(src_h.at[pl.ds(off, _GCH)], mb[p], lsem[p])

      issue(0, 0)

      def step(j, carry):
        for p in range(2):
          @pl.when((j & 1) == p)
          def _():
            @pl.when(j + 1 < its)
            def _():
              issue(j + 1, 1 - p)
            off = base + j * _GCH
            pltpu.make_async_copy(
                src_h.at[pl.ds(off, _GCH)], mb[p], lsem[p]).wait()
            pltpu.sync_copy(mb[p], acc.at[ix[p].at[0]], add=True)
        return carry

      lax.fori_loop(0, its, step, 0)

    @pl.when(c == 0)
    def _():
      body(msg_h)

    @pl.when(c == 1)
    def _():
      body(exd_h)

    plsc.subcore_barrier()

    @pl.when(c == 0)
    def _():
      pltpu.sync_copy(acc.at[pl.ds(r0, _RPT)], nm_o.at[pl.ds(r0, _RPT)])

    @pl.when(c == 1)
    def _():
      pltpu.sync_copy(acc.at[pl.ds(r0, _RPT)], dn_o.at[pl.ds(r0, _RPT)])

  return sk(msg, exd, row, znm)


def _bdot(a, b):
  return jnp.dot(a.astype(jnp.bfloat16), b.astype(jnp.bfloat16),
                 preferred_element_type=jnp.float32)


def _qkv_of(hb, q_w, k_w, v_w):
  q = _bdot(hb, q_w[...])
  kv = jnp.concatenate([
      _bdot(hb, k_w[...]),
      _bdot(hb, v_w[...]),
  ], axis=1)
  return q, kv


def _tc_embed_qkv(x2, table_pad, wq, wk, wv):
  def body(x_ref, t_ref, q_w, k_w, v_w, h_o, q_o, kv_o):
    xi = x_ref[...]
    oh = (xi == lax.broadcasted_iota(jnp.int32, (128, 128), 1)
          ).astype(jnp.float32)
    hb = jnp.dot(oh, t_ref[...], preferred_element_type=jnp.float32, precision=lax.Precision.HIGHEST)
    h_o[...] = hb
    q, kv = _qkv_of(hb, q_w, k_w, v_w)
    q_o[...] = q
    kv_o[...] = kv

  wspec = pl.BlockSpec((128, 128), lambda i: (0, 0))
  hspec = pl.BlockSpec((128, 128), lambda i: (i, 0))
  return pl.pallas_call(
      body,
      grid=(_NBLK,),
      in_specs=[pl.BlockSpec((128, 1), lambda i: (i, 0)),
                wspec, wspec, wspec, wspec],
      out_specs=[hspec, hspec, pl.BlockSpec((128, 256), lambda i: (i, 0))],
      out_shape=[
          jax.ShapeDtypeStruct((_NP, 128), jnp.float32),
          jax.ShapeDtypeStruct((_NP, 128), jnp.float32),
          jax.ShapeDtypeStruct((_NP, 256), jnp.float32),
      ],
  )(x2, table_pad, wq, wk, wv)


def _tc_elen(posr, posc):
  def body(pr, pc, el_o):
    d = pr[...] - pc[...]
    d2 = jnp.sum(d * d, axis=1, keepdims=True)
    el_o[...] = jnp.sqrt(d2)

  espec = pl.BlockSpec((_EB, 128), lambda i: (i, 0))
  return pl.pallas_call(
      body,
      grid=(_EG,),
      in_specs=[espec, espec],
      out_specs=pl.BlockSpec((_EB, 1), lambda i: (i, 0)),
      out_shape=jax.ShapeDtypeStruct((_E, 1), jnp.float32),
  )(posr, posc)


def _tc_edge(elen, qr, kv_g, we_l, offs):
  def body(el_r, q_r, kv_r, w_r, of_r, msg_o, exd_o):
    el = el_r[...]
    a = (el - of_r[...]) * (1.0 / _WIDTH)
    attr = jnp.exp(-0.5 * a * a)
    e = _bdot(attr, w_r[...])
    kvb = kv_r[...]
    prod = q_r[...] * kvb[:, :128] * e
    r16 = lax.broadcasted_iota(jnp.int32, (128, 16), 0) // 16
    c16 = lax.broadcasted_iota(jnp.int32, (128, 16), 1)
    seg = (r16 == c16).astype(jnp.float32)
    s16 = jnp.dot(prod, seg, preferred_element_type=jnp.float32, precision=lax.Precision.HIGHEST) * 0.25
    ex = jnp.exp(s16)
    rt = lax.broadcasted_iota(jnp.int32, (16, 128), 0)
    ct = lax.broadcasted_iota(jnp.int32, (16, 128), 1) // 16
    segt = (rt == ct).astype(jnp.float32)
    exd = jnp.dot(ex, segt, preferred_element_type=jnp.float32, precision=lax.Precision.HIGHEST)
    exd_o[...] = exd
    msg_o[...] = exd * kvb[:, 128:]

  ecnt = elen.shape[0]
  espec = pl.BlockSpec((_EB, 128), lambda i: (i, 0))
  return pl.pallas_call(
      body,
      grid=(ecnt // _EB,),
      in_specs=[
          pl.BlockSpec((_EB, 1), lambda i: (i, 0)),
          espec,
          pl.BlockSpec((_EB, 256), lambda i: (i, 0)),
          pl.BlockSpec((128, 128), lambda i: (0, 0)),
          pl.BlockSpec((1, 128), lambda i: (0, 0)),
      ],
      out_specs=[espec, espec],
      out_shape=[
          jax.ShapeDtypeStruct((ecnt, 128), jnp.float32),
          jax.ShapeDtypeStruct((ecnt, 128), jnp.float32),
      ],
  )(elen, qr, kv_g, we_l, offs)


def _layernorm(t, s_row, b_row):
  mu = jnp.mean(t, axis=-1, keepdims=True)
  var = jnp.mean((t - mu) * (t - mu), axis=-1, keepdims=True)
  return (t - mu) / jnp.sqrt(var + 1e-5) * s_row + b_row


def _tc_update(h, nm, nm2, dnb, dnb2, wo, l1s, l1b, w1, b1, w2, b2, l2s, l2b):
  def body(h_ref, n_r, n2_r, d_r, d2_r, wo_r, l1s_r, l1b_r, w1_r, b1_r,
           w2_r, b2_r, l2s_r, l2b_r, o_ref):
    agg = (n_r[...] + n2_r[...]) / (d_r[...] + d2_r[...] + 1e-9)
    t1 = _layernorm(
        h_ref[...] + _bdot(agg, wo_r[...]), l1s_r[...], l1b_r[...])
    ff = _bdot(
        jnp.maximum(_bdot(t1, w1_r[...]) + b1_r[...], 0.0),
        w2_r[...]) + b2_r[...]
    o_ref[...] = _layernorm(t1 + ff, l2s_r[...], l2b_r[...])

  hspec = pl.BlockSpec((128, 128), lambda i: (i, 0))
  return pl.pallas_call(
      body,
      grid=(_NBLK,),
      in_specs=[
          hspec,
          hspec,
          hspec,
          hspec,
          hspec,
          pl.BlockSpec((128, 128), lambda i: (0, 0)),
          pl.BlockSpec((1, 128), lambda i: (0, 0)),
          pl.BlockSpec((1, 128), lambda i: (0, 0)),
          pl.BlockSpec((128, 256), lambda i: (0, 0)),
          pl.BlockSpec((1, 256), lambda i: (0, 0)),
          pl.BlockSpec((256, 128), lambda i: (0, 0)),
          pl.BlockSpec((1, 128), lambda i: (0, 0)),
          pl.BlockSpec((1, 128), lambda i: (0, 0)),
          pl.BlockSpec((1, 128), lambda i: (0, 0)),
      ],
      out_specs=hspec,
      out_shape=jax.ShapeDtypeStruct((_NP, 128), jnp.float32),
  )(h, nm, nm2, dnb, dnb2, wo, l1s, l1b, w1, b1, w2, b2, l2s, l2b)


def _tc_update_qkv(h, nm, nm2, dnb, dnb2, wo, l1s, l1b, w1, b1, w2, b2,
                   l2s, l2b, wq, wk, wv):
  def body(h_ref, n_r, n2_r, d_r, d2_r, wo_r, l1s_r, l1b_r, w1_r, b1_r,
           w2_r, b2_r, l2s_r, l2b_r, q_w, k_w, v_w, h_o, q_o, kv_o):
    agg = (n_r[...] + n2_r[...]) / (d_r[...] + d2_r[...] + 1e-9)
    t1 = _layernorm(
        h_ref[...] + _bdot(agg, wo_r[...]), l1s_r[...], l1b_r[...])
    ff = _bdot(
        jnp.maximum(_bdot(t1, w1_r[...]) + b1_r[...], 0.0),
        w2_r[...]) + b2_r[...]
    h2 = _layernorm(t1 + ff, l2s_r[...], l2b_r[...])
    h_o[...] = h2
    q, kv = _qkv_of(h2, q_w, k_w, v_w)
    q_o[...] = q
    kv_o[...] = kv

  hspec = pl.BlockSpec((128, 128), lambda i: (i, 0))
  wspec = pl.BlockSpec((128, 128), lambda i: (0, 0))
  rspec = pl.BlockSpec((1, 128), lambda i: (0, 0))
  return pl.pallas_call(
      body,
      grid=(_NBLK,),
      in_specs=[
          hspec, hspec, hspec, hspec, hspec,
          wspec, rspec, rspec,
          pl.BlockSpec((128, 256), lambda i: (0, 0)),
          pl.BlockSpec((1, 256), lambda i: (0, 0)),
          pl.BlockSpec((256, 128), lambda i: (0, 0)),
          rspec, rspec, rspec,
          wspec, wspec, wspec,
      ],
      out_specs=[hspec, hspec, pl.BlockSpec((128, 256), lambda i: (i, 0))],
      out_shape=[
          jax.ShapeDtypeStruct((_NP, 128), jnp.float32),
          jax.ShapeDtypeStruct((_NP, 128), jnp.float32),
          jax.ShapeDtypeStruct((_NP, 256), jnp.float32),
      ],
  )(h, nm, nm2, dnb, dnb2, wo, l1s, l1b, w1, b1, w2, b2, l2s, l2b,
    wq, wk, wv)


def _tc_pool(batch3, h, rw1, rb1r, rw2p, rb2p):
  def body(b_ref, h_ref, w1_r, b1_r, w2_r, b2_r, o_ref, acc):
    i = pl.program_id(0)

    @pl.when(i == 0)
    def _():
      acc[...] = jnp.zeros_like(acc)

    bv = b_ref[0]
    oh = (jnp.broadcast_to(bv, (16, 128))
          == lax.broadcasted_iota(jnp.int32, (16, 128), 0)
          ).astype(jnp.float32)
    acc[...] += jnp.dot(oh, h_ref[...], preferred_element_type=jnp.float32, precision=lax.Precision.HIGHEST)

    @pl.when(i == _NBLK - 1)
    def _():
      z = jnp.maximum(_bdot(acc[...], w1_r[...]) + b1_r[...], 0.0)
      o_ref[...] = _bdot(z, w2_r[...]) + b2_r[...]

  return pl.pallas_call(
      body,
      grid=(_NBLK,),
      in_specs=[
          pl.BlockSpec((1, 1, 128), lambda i: (i, 0, 0)),
          pl.BlockSpec((128, 128), lambda i: (i, 0)),
          pl.BlockSpec((128, 128), lambda i: (0, 0)),
          pl.BlockSpec((1, 128), lambda i: (0, 0)),
          pl.BlockSpec((128, 128), lambda i: (0, 0)),
          pl.BlockSpec((1, 128), lambda i: (0, 0)),
      ],
      out_specs=pl.BlockSpec((16, 128), lambda i: (0, 0)),
      out_shape=jax.ShapeDtypeStruct((16, 128), jnp.float32),
      scratch_shapes=[pltpu.VMEM((16, 128), jnp.float32)],
  )(batch3, h, rw1, rb1r, rw2p, rb2p)


def kernel(pos, x, batch, edge_index, node_table, Wq, Wk, Wv, We, Wo,
           ln1_s, ln1_b, W1, b1, W2, b2, ln2_s, ln2_b, RW1, rb1, RW2, rb2):
  f32 = jnp.float32
  row = edge_index[0].astype(jnp.int32)
  col = edge_index[1].astype(jnp.int32)

  pos128 = jnp.concatenate(
      [pos.astype(f32), jnp.zeros((_N, 125), f32)], axis=1)
  x2 = jnp.concatenate(
      [x.astype(jnp.int32), jnp.zeros((_NP - _N,), jnp.int32)]
  ).reshape(_NP, 1)
  batch3 = jnp.concatenate(
      [batch.astype(jnp.int32), jnp.full((_NP - _N,), 16, jnp.int32)]
  ).reshape(_NBLK, 1, 128)
  table_pad = jnp.concatenate(
      [node_table.astype(f32), jnp.zeros((128 - node_table.shape[0], _D), f32)],
      axis=0)
  offs = jnp.linspace(_CUTON, _CUTOFF, _D, dtype=f32).reshape(1, _D)
  znm = jnp.zeros((_NP, 128), f32)

  posr, posc = _sc_gather([pos128, pos128], [row, col])
  elen = _tc_elen(posr, posc)
  h, q, kv = _tc_embed_qkv(x2, table_pad, Wq[0], Wk[0], Wv[0])

  e2 = 161280
  rowA, rowB = row[:e2], row[e2:]
  colA, colB = col[:e2], col[e2:]
  elenA, elenB = elen[:e2], elen[e2:]

  def edge_phase(q, kv, we_l):
    qrA, kvA = _sc_gather([q, kv], [rowA, colA])
    msgA, exdA = _tc_edge(elenA, qrA, kvA, we_l, offs)
    qrB, kvB = _sc_gather([q, kv], [rowB, colB])
    msgB, exdB = _tc_edge(elenB, qrB, kvB, we_l, offs)
    nmA, dnA = _sc_scatter(msgA, exdA, rowA, znm)
    nmB, dnB = _sc_scatter(msgB, exdB, rowB, znm)
    return nmA, nmB, dnA, dnB

  nmA, nmB, dnA, dnB = edge_phase(q, kv, We[0])
  h, q, kv = _tc_update_qkv(
      h, nmA, nmB, dnA, dnB, Wo[0],
      ln1_s[0].reshape(1, _D), ln1_b[0].reshape(1, _D),
      W1[0], b1[0].reshape(1, 2 * _D), W2[0], b2[0].reshape(1, _D),
      ln2_s[0].reshape(1, _D), ln2_b[0].reshape(1, _D),
      Wq[1], Wk[1], Wv[1])

  nmA, nmB, dnA, dnB = edge_phase(q, kv, We[1])
  h = _tc_update(h, nmA, nmB, dnA, dnB,
                 Wo[1], ln1_s[1].reshape(1, _D), ln1_b[1].reshape(1, _D),
                 W1[1], b1[1].reshape(1, 2 * _D), W2[1],
                 b2[1].reshape(1, _D), ln2_s[1].reshape(1, _D),
                 ln2_b[1].reshape(1, _D))

  rw2p = jnp.concatenate([RW2.astype(f32), jnp.zeros((_D, 127), f32)], axis=1)
  rb2p = jnp.broadcast_to(rb2.astype(f32).reshape(1, 1), (1, 128))
  out = _tc_pool(batch3, h, RW1, rb1.reshape(1, _D), rw2p, rb2p)
  return out[:, 0]
